# baseline (speedup 1.0000x reference)
import functools
import math

import jax
import jax.numpy as jnp
import numpy as np
from jax import lax
from jax.experimental import pallas as pl
from jax.experimental.pallas import tpu as pltpu
from jax.experimental.pallas import tpu_sc as plsc

N_LEVELS = 16
LVL_CHANNELS = 2
BASE_RES = 16
LOG2_HASHMAP = 19
_S = math.log2(2048 / 16) / (N_LEVELS - 1)
_P1 = 2654435761
_P2 = 805459861


def _level_params():
    offsets = []
    offset = 0
    max_params = 2 ** LOG2_HASHMAP
    for i in range(N_LEVELS):
        res_ctor = int(math.ceil(BASE_RES * 2 ** i))
        params = min(max_params, (res_ctor + 1) ** 3)
        params = int(math.ceil(params / 8) * 8)
        offsets.append(offset)
        offset += params
    offsets.append(offset)
    levels = []
    for lvl in range(N_LEVELS):
        scale = float(np.exp2(lvl * _S) * BASE_RES - 1.0)
        resolution = int(np.ceil(scale)) + 1
        hashmap_size = offsets[lvl + 1] - offsets[lvl]
        grid_size = resolution + 1
        hashed = grid_size ** 3 > hashmap_size
        levels.append((scale, grid_size, hashmap_size, hashed, offsets[lvl]))
    return offsets, levels


_OFFSETS, _LEVELS = _level_params()
N_ROWS = _OFFSETS[-1]

NC, NS = 2, 16
NW = NC * NS

BR = 256
NBLK = 220
RT = BR * NBLK
N8P = NS * RT
N_PAD = N8P * 8
assert N_PAD >= N_ROWS


def _make_body(B, P, CHUNK):
    NG = CHUNK // 16
    NCHUNK = P // CHUNK
    ND = CHUNK * 8 // 1024

    def body(in_hbm, emb0_hbm, emb1_hbm, out_hbm, tab_hbm,
             inx_c, iny_c, inz_c, idx_a, idx_b, sub_a, sub_b, fr_a, fr_b,
             rows_a, rows_b, out_v, c0blk, c1blk, tabblk, l0tab, spm_tab,
             sga, sgb, si0, si1, so0, so1):
        cid = lax.axis_index("c")
        sid = lax.axis_index("s")
        wid = sid * NC + cid
        base = wid * P
        iota = lax.iota(jnp.int32, 16)

        tstart = sid * RT
        si_sems = (si0, si1)
        so_sems = (so0, so1)

        def in_dma(b, sub):
            r0 = (tstart + b * BR) * 8
            c0 = pltpu.async_copy(emb0_hbm.at[pl.ds(r0, BR * 8)],
                                  c0blk.at[sub], si_sems[sub])
            c1 = pltpu.async_copy(emb1_hbm.at[pl.ds(r0, BR * 8)],
                                  c1blk.at[sub], si_sems[sub])
            return c0, c1

        def in_wait(b, sub):
            r0 = (tstart + b * BR) * 8
            pltpu.make_async_copy(emb0_hbm.at[pl.ds(r0, BR * 8)],
                                  c0blk.at[sub], si_sems[sub]).wait()
            pltpu.make_async_copy(emb1_hbm.at[pl.ds(r0, BR * 8)],
                                  c1blk.at[sub], si_sems[sub]).wait()

        def out_dma(b, sub):
            r0 = tstart + b * BR
            pltpu.async_copy(tabblk.at[sub],
                             tab_hbm.at[cid, pl.ds(r0, BR), :], so_sems[sub])

        def out_wait(b, sub):
            r0 = tstart + b * BR
            pltpu.make_async_copy(tabblk.at[sub],
                                  tab_hbm.at[cid, pl.ds(r0, BR), :],
                                  so_sems[sub]).wait()

        in_dma(0, 0)
        rows_pat = lax.shift_right_logical(iota, 3)
        cols_pat = iota & 7

        def pair_body(p2i, carry):
            for sub in range(2):
                b = p2i * 2 + sub

                @pl.when(b + 1 < NBLK)
                def _():
                    in_dma(b + 1, sub ^ 1)

                in_wait(b, sub)

                @pl.when(b >= 2)
                def _():
                    out_wait(b - 2, sub)

                def interleave(i, c2, sub=sub):
                    v0 = c0blk[sub, pl.ds(i * 16, 16)]
                    v1 = c1blk[sub, pl.ds(i * 16, 16)]
                    r2 = 2 * i + rows_pat
                    plsc.store_scatter(tabblk.at[sub], [r2, cols_pat], v0)
                    plsc.store_scatter(tabblk.at[sub], [r2, cols_pat + 8],
                                       v1)
                    return c2

                lax.fori_loop(0, BR // 2, interleave, 0, unroll=4)
                out_dma(b, sub)
            return carry

        lax.fori_loop(0, NBLK // 2, pair_body, 0)
        out_wait(NBLK - 2, 0)
        out_wait(NBLK - 1, 1)
        plsc.subcore_barrier()
        pltpu.sync_copy(tab_hbm.at[cid, pl.ds(0, 616), :], l0tab)
        pltpu.sync_copy(tab_hbm.at[cid, pl.ds(615 + sid * 2428, 2428), :],
                        spm_tab.at[pl.ds(sid * 2428, 2428)])
        plsc.subcore_barrier()

        idx_bufs = (idx_a, idx_b)
        sub_bufs = (sub_a, sub_b)
        fr_bufs = (fr_a, fr_b)
        row_bufs = (rows_a, rows_b)
        sg_sems = (sga, sgb)
        tab_sc = tab_hbm.at[cid]

        def pos_of(v, scale):
            x = (v + 1.0) * 0.5
            p = x * scale + 0.5
            pi = p.astype(jnp.int32)
            return p, pi

        def p1_fire(lvl, pb):
            scale, gsz, hsize, hashed, off_l = _LEVELS[lvl]
            idx_v = idx_bufs[pb]
            sub_v = sub_bufs[pb]
            fr_v = fr_bufs[pb]

            def p1(g, c2):
                off = g * 16
                d = g >> 3
                colb = (g & 7) * 128
                xf = inx_c[pl.ds(off, 16)]
                yf = iny_c[pl.ds(off, 16)]
                zf = inz_c[pl.ds(off, 16)]
                px, xi = pos_of(xf, scale)
                py, yi = pos_of(yf, scale)
                pz, zi = pos_of(zf, scale)
                fr_v[g, pl.ds(0, 16)] = px - xi.astype(jnp.float32)
                fr_v[g, pl.ds(16, 16)] = py - yi.astype(jnp.float32)
                fr_v[g, pl.ds(32, 16)] = pz - zi.astype(jnp.float32)
                if hashed:
                    ux = xi.astype(jnp.uint32)
                    uy = yi.astype(jnp.uint32)
                    uz = zi.astype(jnp.uint32)
                    hy0 = uy * jnp.uint32(_P1)
                    hy1 = hy0 + jnp.uint32(_P1)
                    hz0 = uz * jnp.uint32(_P2)
                    hz1 = hz0 + jnp.uint32(_P2)
                    ux1 = ux + jnp.uint32(1)
                    e = [ux ^ hy0, ux ^ hy1, ux1 ^ hy0, ux1 ^ hy1]
                    mask = jnp.uint32(hsize - 1)
                    qoff = 615 if lvl in (1, 2) else 0
                    for c in range(8):
                        h = (e[c >> 1] ^ (hz1 if (c & 1) else hz0)) & mask
                        idx = h.astype(jnp.int32) + off_l
                        idx_v[d, pl.ds(colb + c * 16, 16)] = (
                            lax.shift_right_logical(idx, 3) - qoff)
                        sub_v[g, pl.ds(c * 16, 16)] = idx & 7
                else:
                    qoff = 615 if lvl in (1, 2) else 0
                    sy = yi * gsz
                    sz0 = zi * (gsz * gsz) + off_l
                    sz1 = sz0 + gsz * gsz
                    a00 = xi + sy
                    a = [a00, a00 + gsz, a00 + 1, a00 + gsz + 1]
                    for c in range(8):
                        idx = a[c >> 1] + (sz1 if (c & 1) else sz0)
                        idx_v[d, pl.ds(colb + c * 16, 16)] = (
                            lax.shift_right_logical(idx, 3) - qoff)
                        sub_v[g, pl.ds(c * 16, 16)] = idx & 7
                return c2

            lax.fori_loop(0, NG, p1, 0)

            if lvl == 0:
                return

            rows_v = row_bufs[pb]
            sg = sg_sems[pb]

            gsrc = spm_tab if lvl in (1, 2) else tab_sc

            def fire(dd, c2):
                pltpu.async_copy(gsrc.at[idx_v.at[dd]], rows_v.at[dd], sg)
                return c2

            lax.fori_loop(0, ND, fire, 0)

        def drain_p2(lvl, pb):
            idx_v = idx_bufs[pb]
            sub_v = sub_bufs[pb]
            fr_v = fr_bufs[pb]
            rows_v = row_bufs[pb]
            sg = sg_sems[pb]

            gsrc = spm_tab if lvl in (1, 2) else tab_sc

            def drain(dd, c2):
                pltpu.make_async_copy(gsrc.at[idx_v.at[dd]],
                                      rows_v.at[dd], sg).wait()
                return c2

            if lvl != 0:
                lax.fori_loop(0, ND, drain, 0)

            def p2(g, c2):
                d = g >> 3
                colb = (g & 7) * 128
                dsp = jnp.full((16,), d, jnp.int32)
                fx = fr_v[g, pl.ds(0, 16)]
                fy = fr_v[g, pl.ds(16, 16)]
                fz = fr_v[g, pl.ds(32, 16)]
                wx0 = 1.0 - fx
                wy0 = 1.0 - fy
                wz0 = 1.0 - fz
                w = [wx0 * wy0, wx0 * fy, fx * wy0, fx * fy]
                acc0 = jnp.zeros((16,), jnp.float32)
                acc1 = jnp.zeros((16,), jnp.float32)
                for c in range(8):
                    wc = w[c >> 1] * (fz if (c & 1) else wz0)
                    wq = sub_v[g, pl.ds(c * 16, 16)]
                    if lvl == 0:
                        qv = idx_v[d, pl.ds(colb + c * 16, 16)]
                        v0 = plsc.load_gather(l0tab, [qv, wq])
                        v1 = plsc.load_gather(l0tab, [qv, wq + 8])
                    else:
                        pos = colb + c * 16 + iota
                        v0 = plsc.load_gather(rows_v, [dsp, pos, wq])
                        v1 = plsc.load_gather(rows_v, [dsp, pos, wq + 8])
                    acc0 = acc0 + wc * v0
                    acc1 = acc1 + wc * v1
                out_v[2 * lvl, pl.ds(g * 16, 16)] = acc0
                out_v[2 * lvl + 1, pl.ds(g * 16, 16)] = acc1
                return c2

            lax.fori_loop(0, NG, p2, 0)

        def chunk_body(ci, carry):
            cbase = ci * CHUNK
            pltpu.sync_copy(in_hbm.at[pl.ds(base + cbase, CHUNK)], inx_c)
            pltpu.sync_copy(in_hbm.at[pl.ds(B + base + cbase, CHUNK)],
                            iny_c)
            pltpu.sync_copy(in_hbm.at[pl.ds(2 * B + base + cbase, CHUNK)],
                            inz_c)
            p1_fire(0, 0)
            for lvl in range(N_LEVELS):
                if lvl + 1 < N_LEVELS:
                    p1_fire(lvl + 1, (lvl + 1) & 1)
                drain_p2(lvl, lvl & 1)
            pltpu.sync_copy(out_v,
                            out_hbm.at[:, pl.ds(base + cbase, CHUNK)])
            return carry

        lax.fori_loop(0, NCHUNK, chunk_body, 0)

    return body


@jax.jit
def kernel(inputs, embeddings):
    B = inputs.shape[0]
    assert B % (NW * 16) == 0
    P = B // NW
    CHUNK = min(128, P)
    NG = CHUNK // 16
    ND = CHUNK * 8 // 1024
    mesh = plsc.VectorSubcoreMesh(core_axis_name="c", subcore_axis_name="s")
    f = pl.kernel(
        _make_body(B, P, CHUNK),
        out_type=(
            jax.ShapeDtypeStruct((N_LEVELS * LVL_CHANNELS, B), jnp.float32),
            jax.ShapeDtypeStruct((NC, N8P, 16), jnp.float32),
        ),
        mesh=mesh,
        compiler_params=pltpu.CompilerParams(
            needs_layout_passes=False, use_tc_tiling_on_sc=False),
        scratch_types=[
            pltpu.VMEM((CHUNK,), jnp.float32),
            pltpu.VMEM((CHUNK,), jnp.float32),
            pltpu.VMEM((CHUNK,), jnp.float32),
            pltpu.VMEM((ND, 1024), jnp.int32),
            pltpu.VMEM((ND, 1024), jnp.int32),
            pltpu.VMEM((NG, 128), jnp.int32),
            pltpu.VMEM((NG, 128), jnp.int32),
            pltpu.VMEM((NG, 48), jnp.float32),
            pltpu.VMEM((NG, 48), jnp.float32),
            pltpu.VMEM((ND, 1024, 16), jnp.float32),
            pltpu.VMEM((ND, 1024, 16), jnp.float32),
            pltpu.VMEM((N_LEVELS * LVL_CHANNELS, CHUNK), jnp.float32),
            pltpu.VMEM((2, BR * 8), jnp.float32),
            pltpu.VMEM((2, BR * 8), jnp.float32),
            pltpu.VMEM((2, BR, 16), jnp.float32),
            pltpu.VMEM((616, 16), jnp.float32),
            pltpu.VMEM_SHARED((38848, 16), jnp.float32),
            pltpu.SemaphoreType.DMA,
            pltpu.SemaphoreType.DMA,
            pltpu.SemaphoreType.DMA,
            pltpu.SemaphoreType.DMA,
            pltpu.SemaphoreType.DMA,
            pltpu.SemaphoreType.DMA,
        ],
    )
    xt = jnp.transpose(inputs).reshape(-1)
    pad = N_PAD - N_ROWS
    emb0 = jnp.concatenate([embeddings[:, 0],
                            jnp.zeros((pad,), jnp.float32)])
    emb1 = jnp.concatenate([embeddings[:, 1],
                            jnp.zeros((pad,), jnp.float32)])
    out, _ = f(xt, emb0, emb1)
    return jnp.transpose(out)


# speedup vs baseline: 1.0785x; 1.0785x over previous
import functools
import math

import jax
import jax.numpy as jnp
import numpy as np
from jax import lax
from jax.experimental import pallas as pl
from jax.experimental.pallas import tpu as pltpu
from jax.experimental.pallas import tpu_sc as plsc

N_LEVELS = 16
LVL_CHANNELS = 2
BASE_RES = 16
LOG2_HASHMAP = 19
_S = math.log2(2048 / 16) / (N_LEVELS - 1)
_P1 = 2654435761
_P2 = 805459861


def _level_params():
    offsets = []
    offset = 0
    max_params = 2 ** LOG2_HASHMAP
    for i in range(N_LEVELS):
        res_ctor = int(math.ceil(BASE_RES * 2 ** i))
        params = min(max_params, (res_ctor + 1) ** 3)
        params = int(math.ceil(params / 8) * 8)
        offsets.append(offset)
        offset += params
    offsets.append(offset)
    levels = []
    for lvl in range(N_LEVELS):
        scale = float(np.exp2(lvl * _S) * BASE_RES - 1.0)
        resolution = int(np.ceil(scale)) + 1
        hashmap_size = offsets[lvl + 1] - offsets[lvl]
        grid_size = resolution + 1
        hashed = grid_size ** 3 > hashmap_size
        levels.append((scale, grid_size, hashmap_size, hashed, offsets[lvl]))
    return offsets, levels


_OFFSETS, _LEVELS = _level_params()
N_ROWS = _OFFSETS[-1]

NC, NS = 2, 16
NW = NC * NS

BR = 256
NBLK = 220
RT = BR * NBLK
N8P = NS * RT
N_PAD = N8P * 8
assert N_PAD >= N_ROWS


def _make_body(B, P, CHUNK):
    NG = CHUNK // 16
    NCHUNK = P // CHUNK
    ND = CHUNK * 8 // 1024

    def body(in_hbm, emb0_hbm, emb1_hbm, out_hbm, tab_hbm,
             inx_c, iny_c, inz_c, idx_a, idx_b, sub_a, sub_b, fr_a, fr_b,
             rows_a, rows_b, out_v, c0blk, c1blk, tabblk, l0tab, spm_tab,
             sga, sgb, si0, si1, so0, so1):
        cid = lax.axis_index("c")
        sid = lax.axis_index("s")
        wid = sid * NC + cid
        base = wid * P
        iota = lax.iota(jnp.int32, 16)

        tstart = sid * RT
        si_sems = (si0, si1)
        so_sems = (so0, so1)

        def in_dma(b, sub):
            r0 = (tstart + b * BR) * 8
            c0 = pltpu.async_copy(emb0_hbm.at[pl.ds(r0, BR * 8)],
                                  c0blk.at[sub], si_sems[sub])
            c1 = pltpu.async_copy(emb1_hbm.at[pl.ds(r0, BR * 8)],
                                  c1blk.at[sub], si_sems[sub])
            return c0, c1

        def in_wait(b, sub):
            r0 = (tstart + b * BR) * 8
            pltpu.make_async_copy(emb0_hbm.at[pl.ds(r0, BR * 8)],
                                  c0blk.at[sub], si_sems[sub]).wait()
            pltpu.make_async_copy(emb1_hbm.at[pl.ds(r0, BR * 8)],
                                  c1blk.at[sub], si_sems[sub]).wait()

        def out_dma(b, sub):
            r0 = tstart + b * BR
            pltpu.async_copy(tabblk.at[sub],
                             tab_hbm.at[cid, pl.ds(r0, BR), :], so_sems[sub])

        def out_wait(b, sub):
            r0 = tstart + b * BR
            pltpu.make_async_copy(tabblk.at[sub],
                                  tab_hbm.at[cid, pl.ds(r0, BR), :],
                                  so_sems[sub]).wait()

        in_dma(0, 0)
        rows_pat = lax.shift_right_logical(iota, 3)
        cols_pat = iota & 7

        def pair_body(p2i, carry):
            for sub in range(2):
                b = p2i * 2 + sub

                @pl.when(b + 1 < NBLK)
                def _():
                    in_dma(b + 1, sub ^ 1)

                in_wait(b, sub)

                @pl.when(b >= 2)
                def _():
                    out_wait(b - 2, sub)

                def interleave(i, c2, sub=sub):
                    v0 = c0blk[sub, pl.ds(i * 16, 16)]
                    v1 = c1blk[sub, pl.ds(i * 16, 16)]
                    r2 = 2 * i + rows_pat
                    plsc.store_scatter(tabblk.at[sub], [r2, cols_pat], v0)
                    plsc.store_scatter(tabblk.at[sub], [r2, cols_pat + 8],
                                       v1)
                    return c2

                lax.fori_loop(0, BR // 2, interleave, 0, unroll=4)
                out_dma(b, sub)
            return carry

        lax.fori_loop(0, NBLK // 2, pair_body, 0)
        out_wait(NBLK - 2, 0)
        out_wait(NBLK - 1, 1)
        plsc.subcore_barrier()
        pltpu.sync_copy(tab_hbm.at[cid, pl.ds(0, 616), :], l0tab)
        pltpu.sync_copy(tab_hbm.at[cid, pl.ds(615 + sid * 281, 281), :],
                        spm_tab.at[pl.ds(sid * 281, 281)])
        plsc.subcore_barrier()

        idx_bufs = (idx_a, idx_b)
        sub_bufs = (sub_a, sub_b)
        fr_bufs = (fr_a, fr_b)
        row_bufs = (rows_a, rows_b)
        sg_sems = (sga, sgb)
        tab_sc = tab_hbm.at[cid]

        def pos_of(v, scale):
            x = (v + 1.0) * 0.5
            p = x * scale + 0.5
            pi = p.astype(jnp.int32)
            return p, pi

        def p1_fire(lvl, pb):
            scale, gsz, hsize, hashed, off_l = _LEVELS[lvl]
            idx_v = idx_bufs[pb]
            sub_v = sub_bufs[pb]
            fr_v = fr_bufs[pb]

            def p1(g, c2):
                off = g * 16
                d = g >> 3
                colb = (g & 7) * 128
                xf = inx_c[pl.ds(off, 16)]
                yf = iny_c[pl.ds(off, 16)]
                zf = inz_c[pl.ds(off, 16)]
                px, xi = pos_of(xf, scale)
                py, yi = pos_of(yf, scale)
                pz, zi = pos_of(zf, scale)
                fr_v[g, pl.ds(0, 16)] = px - xi.astype(jnp.float32)
                fr_v[g, pl.ds(16, 16)] = py - yi.astype(jnp.float32)
                fr_v[g, pl.ds(32, 16)] = pz - zi.astype(jnp.float32)
                if hashed:
                    ux = xi.astype(jnp.uint32)
                    uy = yi.astype(jnp.uint32)
                    uz = zi.astype(jnp.uint32)
                    hy0 = uy * jnp.uint32(_P1)
                    hy1 = hy0 + jnp.uint32(_P1)
                    hz0 = uz * jnp.uint32(_P2)
                    hz1 = hz0 + jnp.uint32(_P2)
                    ux1 = ux + jnp.uint32(1)
                    e = [ux ^ hy0, ux ^ hy1, ux1 ^ hy0, ux1 ^ hy1]
                    mask = jnp.uint32(hsize - 1)
                    qoff = 615 if lvl == 1 else 0
                    for c in range(8):
                        h = (e[c >> 1] ^ (hz1 if (c & 1) else hz0)) & mask
                        idx = h.astype(jnp.int32) + off_l
                        idx_v[d, pl.ds(colb + c * 16, 16)] = (
                            lax.shift_right_logical(idx, 3) - qoff)
                        sub_v[g, pl.ds(c * 16, 16)] = idx & 7
                else:
                    qoff = 615 if lvl == 1 else 0
                    sy = yi * gsz
                    sz0 = zi * (gsz * gsz) + off_l
                    sz1 = sz0 + gsz * gsz
                    a00 = xi + sy
                    a = [a00, a00 + gsz, a00 + 1, a00 + gsz + 1]
                    for c in range(8):
                        idx = a[c >> 1] + (sz1 if (c & 1) else sz0)
                        idx_v[d, pl.ds(colb + c * 16, 16)] = (
                            lax.shift_right_logical(idx, 3) - qoff)
                        sub_v[g, pl.ds(c * 16, 16)] = idx & 7
                return c2

            lax.fori_loop(0, NG, p1, 0)

            if lvl == 0:
                return

            rows_v = row_bufs[pb]
            sg = sg_sems[pb]

            gsrc = spm_tab if lvl == 1 else tab_sc

            def fire(dd, c2):
                pltpu.async_copy(gsrc.at[idx_v.at[dd]], rows_v.at[dd], sg)
                return c2

            lax.fori_loop(0, ND, fire, 0)

        def drain_p2(lvl, pb):
            idx_v = idx_bufs[pb]
            sub_v = sub_bufs[pb]
            fr_v = fr_bufs[pb]
            rows_v = row_bufs[pb]
            sg = sg_sems[pb]

            gsrc = spm_tab if lvl == 1 else tab_sc

            def drain(dd, c2):
                pltpu.make_async_copy(gsrc.at[idx_v.at[dd]],
                                      rows_v.at[dd], sg).wait()
                return c2

            if lvl != 0:
                lax.fori_loop(0, ND, drain, 0)

            def p2(g, c2):
                d = g >> 3
                colb = (g & 7) * 128
                dsp = jnp.full((16,), d, jnp.int32)
                fx = fr_v[g, pl.ds(0, 16)]
                fy = fr_v[g, pl.ds(16, 16)]
                fz = fr_v[g, pl.ds(32, 16)]
                wx0 = 1.0 - fx
                wy0 = 1.0 - fy
                wz0 = 1.0 - fz
                w = [wx0 * wy0, wx0 * fy, fx * wy0, fx * fy]
                acc0 = jnp.zeros((16,), jnp.float32)
                acc1 = jnp.zeros((16,), jnp.float32)
                for c in range(8):
                    wc = w[c >> 1] * (fz if (c & 1) else wz0)
                    wq = sub_v[g, pl.ds(c * 16, 16)]
                    if lvl == 0:
                        qv = idx_v[d, pl.ds(colb + c * 16, 16)]
                        v0 = plsc.load_gather(l0tab, [qv, wq])
                        v1 = plsc.load_gather(l0tab, [qv, wq + 8])
                    else:
                        pos = colb + c * 16 + iota
                        v0 = plsc.load_gather(rows_v, [dsp, pos, wq])
                        v1 = plsc.load_gather(rows_v, [dsp, pos, wq + 8])
                    acc0 = acc0 + wc * v0
                    acc1 = acc1 + wc * v1
                out_v[2 * lvl, pl.ds(g * 16, 16)] = acc0
                out_v[2 * lvl + 1, pl.ds(g * 16, 16)] = acc1
                return c2

            lax.fori_loop(0, NG, p2, 0)

        def chunk_body(ci, carry):
            cbase = ci * CHUNK
            pltpu.sync_copy(in_hbm.at[pl.ds(base + cbase, CHUNK)], inx_c)
            pltpu.sync_copy(in_hbm.at[pl.ds(B + base + cbase, CHUNK)],
                            iny_c)
            pltpu.sync_copy(in_hbm.at[pl.ds(2 * B + base + cbase, CHUNK)],
                            inz_c)
            p1_fire(0, 0)
            for lvl in range(N_LEVELS):
                if lvl + 1 < N_LEVELS:
                    p1_fire(lvl + 1, (lvl + 1) & 1)
                drain_p2(lvl, lvl & 1)
            pltpu.sync_copy(out_v,
                            out_hbm.at[:, pl.ds(base + cbase, CHUNK)])
            return carry

        lax.fori_loop(0, NCHUNK, chunk_body, 0)

    return body


@jax.jit
def kernel(inputs, embeddings):
    B = inputs.shape[0]
    assert B % (NW * 16) == 0
    P = B // NW
    CHUNK = min(256, P)
    NG = CHUNK // 16
    ND = CHUNK * 8 // 1024
    mesh = plsc.VectorSubcoreMesh(core_axis_name="c", subcore_axis_name="s")
    f = pl.kernel(
        _make_body(B, P, CHUNK),
        out_type=(
            jax.ShapeDtypeStruct((N_LEVELS * LVL_CHANNELS, B), jnp.float32),
            jax.ShapeDtypeStruct((NC, N8P, 16), jnp.float32),
        ),
        mesh=mesh,
        compiler_params=pltpu.CompilerParams(
            needs_layout_passes=False, use_tc_tiling_on_sc=False),
        scratch_types=[
            pltpu.VMEM((CHUNK,), jnp.float32),
            pltpu.VMEM((CHUNK,), jnp.float32),
            pltpu.VMEM((CHUNK,), jnp.float32),
            pltpu.VMEM((ND, 1024), jnp.int32),
            pltpu.VMEM((ND, 1024), jnp.int32),
            pltpu.VMEM((NG, 128), jnp.int32),
            pltpu.VMEM((NG, 128), jnp.int32),
            pltpu.VMEM((NG, 48), jnp.float32),
            pltpu.VMEM((NG, 48), jnp.float32),
            pltpu.VMEM((ND, 1024, 16), jnp.float32),
            pltpu.VMEM((ND, 1024, 16), jnp.float32),
            pltpu.VMEM((N_LEVELS * LVL_CHANNELS, CHUNK), jnp.float32),
            pltpu.VMEM((2, BR * 8), jnp.float32),
            pltpu.VMEM((2, BR * 8), jnp.float32),
            pltpu.VMEM((2, BR, 16), jnp.float32),
            pltpu.VMEM((616, 16), jnp.float32),
            pltpu.VMEM_SHARED((4496, 16), jnp.float32),
            pltpu.SemaphoreType.DMA,
            pltpu.SemaphoreType.DMA,
            pltpu.SemaphoreType.DMA,
            pltpu.SemaphoreType.DMA,
            pltpu.SemaphoreType.DMA,
            pltpu.SemaphoreType.DMA,
        ],
    )
    xt = jnp.transpose(inputs).reshape(-1)
    pad = N_PAD - N_ROWS
    emb0 = jnp.concatenate([embeddings[:, 0],
                            jnp.zeros((pad,), jnp.float32)])
    emb1 = jnp.concatenate([embeddings[:, 1],
                            jnp.zeros((pad,), jnp.float32)])
    out, _ = f(xt, emb0, emb1)
    return jnp.transpose(out)
